# Initial kernel scaffold; baseline (speedup 1.0000x reference)
import jax
import jax.numpy as jnp
from jax.experimental import pallas as pl


def kernel(emb, proto_w):
    raise NotImplementedError("write your pallas kernel here")


# speedup vs baseline: 119.0530x; 119.0530x over previous
import functools

import jax
import jax.numpy as jnp
from jax.experimental import pallas as pl

_B = 512
_N = 68
_K = 15
_D = 128


def _cos_classifier_body(x_ref, p_ref, xa_ref, pa_ref, out_ref):
    x = x_ref[...]
    p = p_ref[...]

    xn2 = jnp.sum(x * x, axis=1, keepdims=True)
    pn2 = jnp.sum(p * p, axis=1, keepdims=True)
    xs = x * jax.lax.rsqrt(jnp.maximum(xn2, 1e-24))
    ps = p * jax.lax.rsqrt(jnp.maximum(pn2, 1e-24))

    d2 = jnp.zeros((_K, _B, _N), dtype=jnp.float32)
    for c in range(3):
        diff = xa_ref[c][:, :, None] - pa_ref[c][:, None, :]
        d2 = d2 + diff * diff
    ang = jnp.sqrt(d2)

    t = ang * (1.0 / 200.0)
    m = jnp.max(t, axis=0, keepdims=True)
    e = jnp.exp(t - m)
    s = jnp.sum(e, axis=0, keepdims=True)
    w2 = e * (15.0 / s)

    acc = jnp.zeros((_B, _N), dtype=jnp.float32)
    for k in range(_K):
        xk = xs[:, k * _D:(k + 1) * _D]
        pk = ps[:, k * _D:(k + 1) * _D]
        sk = jax.lax.dot_general(
            xk, pk,
            dimension_numbers=(((1,), (1,)), ((), ())),
            preferred_element_type=jnp.float32,
        )
        acc = acc + w2[k] * sk
    out_ref[...] = acc * 16.0


@jax.jit
def kernel(emb, proto_w):
    x = emb[:, : _K * _D]
    p = proto_w[:, : _K * _D]
    xa = jnp.transpose(emb[:, _K * _D:].reshape(_B, _K, 3), (2, 1, 0))
    pa = jnp.transpose(proto_w[:, _K * _D:].reshape(_N, _K, 3), (2, 1, 0))
    return pl.pallas_call(
        _cos_classifier_body,
        out_shape=jax.ShapeDtypeStruct((_B, _N), jnp.float32),
    )(x, p, xa, pa)


# speedup vs baseline: 128.6361x; 1.0805x over previous
import jax
import jax.numpy as jnp
from jax.experimental import pallas as pl

_B = 512
_N = 68
_NP = 72
_K = 15
_D = 128
_F = _K * _D
_CA = 8


def _cos_classifier_body(emb_ref, pw_ref, xa_ref, pa_ref, out_ref):
    x = emb_ref[:, :_F]
    p = pw_ref[:, :_F]
    ones_f = jnp.ones((1, _F), dtype=jnp.float32)

    xn2 = jax.lax.dot_general(ones_f, x * x, (((1,), (1,)), ((), ())),
                              preferred_element_type=jnp.float32)
    pn2 = jax.lax.dot_general(p * p, ones_f, (((1,), (1,)), ((), ())),
                              preferred_element_type=jnp.float32)
    rx = jax.lax.rsqrt(jnp.maximum(xn2, 1e-24))
    rp = jax.lax.rsqrt(jnp.maximum(pn2, 1e-24))

    d2 = jax.lax.dot_general(pa_ref[...], xa_ref[...],
                             (((1,), (1,)), ((), ())),
                             preferred_element_type=jnp.float32)
    t = jnp.sqrt(jnp.maximum(d2.reshape(_K, _NP, _B), 0.0))
    m = jnp.max(t, axis=0, keepdims=True)
    e = jnp.exp(t - m)
    s = jnp.sum(e, axis=0, keepdims=True)
    w2 = e * ((240.0 * rx[None]) / s)

    acc = jnp.zeros((_N, _B), dtype=jnp.float32)
    for k in range(_K):
        sk = jax.lax.dot_general(
            p[:, k * _D:(k + 1) * _D], x[:, k * _D:(k + 1) * _D],
            dimension_numbers=(((1,), (1,)), ((), ())),
            preferred_element_type=jnp.float32,
        )
        acc = acc + w2[k, :_N, :] * sk
    out_ref[...] = jnp.transpose(acc * rp)


@jax.jit
def kernel(emb, proto_w):
    u = emb[:, _F:].reshape(_B, _K, 3) * (1.0 / 200.0)
    v = proto_w[:, _F:].reshape(_N, _K, 3) * (1.0 / 200.0)
    zpad = jnp.zeros((_B, _K, _CA - 5), jnp.float32)
    xa = jnp.concatenate(
        [u, jnp.sum(u * u, axis=2, keepdims=True),
         jnp.ones((_B, _K, 1), jnp.float32), zpad], axis=2)
    xa = xa.reshape(_B, _K * _CA)
    pv = jnp.concatenate(
        [-2.0 * v, jnp.ones((_N, _K, 1), jnp.float32),
         jnp.sum(v * v, axis=2, keepdims=True),
         jnp.zeros((_N, _K, _CA - 5), jnp.float32)], axis=2)
    pv = jnp.pad(pv, ((0, _NP - _N), (0, 0), (0, 0)))
    pa = jnp.einsum('nkc,kl->knlc', pv, jnp.eye(_K, dtype=jnp.float32))
    pa = pa.reshape(_K * _NP, _K * _CA)
    return pl.pallas_call(
        _cos_classifier_body,
        out_shape=jax.ShapeDtypeStruct((_B, _N), jnp.float32),
    )(emb, proto_w, xa, pa)
